# Initial kernel scaffold; baseline (speedup 1.0000x reference)
import jax
import jax.numpy as jnp
from jax.experimental import pallas as pl


def kernel(x, tables):
    raise NotImplementedError("write your pallas kernel here")


# speedup vs baseline: 3.2393x; 3.2393x over previous
import functools

import jax
import jax.numpy as jnp
import numpy as np
from jax import lax
from jax.experimental import pallas as pl
from jax.experimental.pallas import tpu as pltpu
from jax.experimental.pallas import tpu_sc as plsc

F = 26
VOCAB = 100000
DIM = 32
B = 4096
L = 20

N = B * L * F
NC, NS = 2, 16
NW = NC * NS
PER_W = N // NW
SUB = 128
ROWS = 13
CHUNK = ROWS * SUB
NCH = PER_W // CHUNK

_OFFS = ((np.arange(CHUNK, dtype=np.int64) % F) * VOCAB).astype(np.int32)
_OFFS = _OFFS.reshape(ROWS, SUB)

_mesh = plsc.VectorSubcoreMesh(core_axis_name="c", subcore_axis_name="s")


@functools.partial(
    pl.kernel,
    mesh=_mesh,
    out_type=jax.ShapeDtypeStruct((N, DIM), jnp.float32),
    scratch_types=[
        pltpu.VMEM((CHUNK,), jnp.int32),
        pltpu.VMEM((ROWS, SUB), jnp.int32),
        pltpu.VMEM((ROWS, SUB), jnp.int32),
        pltpu.VMEM((CHUNK, DIM), jnp.float32),
        pltpu.SemaphoreType.DMA,
    ],
    compiler_params=pltpu.CompilerParams(use_tc_tiling_on_sc=False),
)
def _gather(tables_hbm, x_hbm, offs_hbm, out_hbm, raw_v, idx_v, offs_v, rows_v, sem):
    wid = lax.axis_index("s") * NC + lax.axis_index("c")
    pltpu.sync_copy(offs_hbm, offs_v)

    def chunk_body(g, carry):
        n0 = wid * PER_W + g * CHUNK
        pltpu.sync_copy(x_hbm.at[pl.ds(n0, CHUNK)], raw_v)
        for r in range(ROWS):
            for s in range(SUB // 16):
                sl = pl.ds(s * 16, 16)
                idx_v[r, sl] = raw_v[pl.ds(r * SUB + s * 16, 16)] + offs_v[r, sl]
        cps = []
        for r in range(ROWS):
            cp = pltpu.make_async_copy(
                tables_hbm.at[idx_v.at[r]],
                rows_v.at[pl.ds(r * SUB, SUB)],
                sem,
            )
            cp.start()
            cps.append(cp)
        for cp in cps:
            cp.wait()
        pltpu.sync_copy(rows_v, out_hbm.at[pl.ds(n0, CHUNK)])
        return carry

    lax.fori_loop(0, NCH, chunk_body, 0)


def kernel(x, tables):
    tables_flat = tables.reshape(F * VOCAB, DIM)
    x_flat = x.reshape(N)
    offs = jnp.asarray(_OFFS)
    out = _gather(tables_flat, x_flat, offs)
    return out.reshape(B, L, F * DIM)


# speedup vs baseline: 3.3255x; 1.0266x over previous
import functools

import jax
import jax.numpy as jnp
import numpy as np
from jax import lax
from jax.experimental import pallas as pl
from jax.experimental.pallas import tpu as pltpu
from jax.experimental.pallas import tpu_sc as plsc

F = 26
VOCAB = 100000
DIM = 32
B = 4096
L = 20

N = B * L * F
NC, NS = 2, 16
NW = NC * NS
PER_W = N // NW
SUB = 128
ROWS = 13
CHUNK = ROWS * SUB
NCH = PER_W // CHUNK

_OFFS = ((np.arange(CHUNK, dtype=np.int64) % F) * VOCAB).astype(np.int32)
_OFFS = _OFFS.reshape(ROWS, SUB)

_mesh = plsc.VectorSubcoreMesh(core_axis_name="c", subcore_axis_name="s")


@functools.partial(
    pl.kernel,
    mesh=_mesh,
    out_type=jax.ShapeDtypeStruct((N, DIM), jnp.float32),
    scratch_types=[
        pltpu.VMEM((CHUNK,), jnp.int32),
        pltpu.VMEM((CHUNK,), jnp.int32),
        pltpu.VMEM((ROWS, SUB), jnp.int32),
        pltpu.VMEM((ROWS, SUB), jnp.int32),
        pltpu.VMEM((ROWS, SUB), jnp.int32),
        pltpu.VMEM((CHUNK, DIM), jnp.float32),
        pltpu.VMEM((CHUNK, DIM), jnp.float32),
        pltpu.SemaphoreType.DMA,
        pltpu.SemaphoreType.DMA,
        pltpu.SemaphoreType.DMA,
        pltpu.SemaphoreType.DMA,
        pltpu.SemaphoreType.DMA,
        pltpu.SemaphoreType.DMA,
    ],
    compiler_params=pltpu.CompilerParams(use_tc_tiling_on_sc=False),
)
def _gather(tables_hbm, x_hbm, offs_hbm, out_hbm,
            raw0, raw1, idx0, idx1, offs_v, rows0, rows1,
            si0, si1, sg0, sg1, sw0, sw1):
    raw = (raw0, raw1)
    idx = (idx0, idx1)
    rows = (rows0, rows1)
    si = (si0, si1)
    sg = (sg0, sg1)
    sw = (sw0, sw1)

    wid = lax.axis_index("s") * NC + lax.axis_index("c")
    base = wid * PER_W
    pltpu.sync_copy(offs_hbm, offs_v)

    def idx_cp(g, p):
        return pltpu.make_async_copy(
            x_hbm.at[pl.ds(base + g * CHUNK, CHUNK)], raw[p], si[p])

    def gather_cps(p):
        return [
            pltpu.make_async_copy(
                tables_hbm.at[idx[p].at[r]],
                rows[p].at[pl.ds(r * SUB, SUB)],
                sg[p],
            )
            for r in range(ROWS)
        ]

    def write_cp(g, p):
        return pltpu.make_async_copy(
            rows[p], out_hbm.at[pl.ds(base + g * CHUNK, CHUNK)], sw[p])

    def compute_gidx(p):
        for r in range(ROWS):
            for s in range(SUB // 16):
                sl = pl.ds(s * 16, 16)
                idx[p][r, sl] = raw[p][pl.ds(r * SUB + s * 16, 16)] + offs_v[r, sl]

    def step(g, p, *, first=False, second=False, last=False):
        idx_cp(g, p).wait()
        compute_gidx(p)
        if not last:
            idx_cp(g + 1, 1 - p).start()
        if not (first or second):
            write_cp(g - 2, p).wait()
        for cp in gather_cps(p):
            cp.start()
        if not first:
            for cp in gather_cps(1 - p):
                cp.wait()
            write_cp(g - 1, 1 - p).start()

    idx_cp(0, 0).start()
    step(0, 0, first=True)
    step(1, 1, second=True)

    def pair(k, carry):
        g = 2 * k
        step(g, 0)
        step(g + 1, 1)
        return carry

    lax.fori_loop(1, NCH // 2 - 1, pair, 0)

    step(NCH - 2, 0)
    step(NCH - 1, 1, last=True)
    for cp in gather_cps(1):
        cp.wait()
    write_cp(NCH - 1, 1).start()
    write_cp(NCH - 2, 0).wait()
    write_cp(NCH - 1, 1).wait()


def kernel(x, tables):
    tables_flat = tables.reshape(F * VOCAB, DIM)
    x_flat = x.reshape(N)
    offs = jnp.asarray(_OFFS)
    out = _gather(tables_flat, x_flat, offs)
    return out.reshape(B, L, F * DIM)
